# baseline (speedup 1.0000x reference)
import functools

import jax
import jax.numpy as jnp
from jax import lax
from jax.experimental import pallas as pl
from jax.experimental.pallas import tpu as pltpu
from jax.experimental.pallas import tpu_sc as plsc

N = 100000
E = 1600000
IN_DIM = 2
EDGE_DIM = 16
NODE_EMB = 32
EDGE_EMB = 8
LAYERS = 4
NUM_CAND = 10000
NUM_GRAPHS = 64
CAT_DIM = 2 * NODE_EMB + EDGE_EMB
OUT_DIM = NODE_EMB + EDGE_EMB

NC = 2
NS = 16
NW = NC * NS

CHUNK = 128
EP = 1638400
NP = 100096
EPW = EP // NW
GG = 5
GEDGES = GG * CHUNK
NGRP_G = EPW // GEDGES
EPT = EP // NS
GS = 8
SEDGES = GS * CHUNK
NGRP_S = EPT // SEDGES
ROWS_PT = NP // NS
HALF = NODE_EMB // NC
QCOL = HALF // 2

CPAD = 10240
CPW = CPAD // NW
CSTEPS = CPW // 16

def _sc_mesh():
    return plsc.VectorSubcoreMesh(core_axis_name="c", subcore_axis_name="s")


_SC_PARAMS = pltpu.CompilerParams(use_tc_tiling_on_sc=False,
                                  needs_layout_passes=False)


def _gather_body(h_hbm, src2_hbm, dst2_hbm, hs_out, hd_out,
                 sidx, didx, hs_blk, hd_blk, sem):
    c = lax.axis_index("c")
    s = lax.axis_index("s")
    wid = s * NC + c

    def body(g, carry):
        cbase = wid * (EPW // CHUNK) + g * GG
        ebase = wid * EPW + g * GEDGES
        pltpu.sync_copy(src2_hbm.at[pl.ds(cbase, GG)], sidx)
        pltpu.sync_copy(dst2_hbm.at[pl.ds(cbase, GG)], didx)
        cps = []
        for k in range(GG):
            cps.append(pltpu.async_copy(
                h_hbm.at[sidx.at[k]],
                hs_blk.at[pl.ds(k * CHUNK, CHUNK)], sem))
            cps.append(pltpu.async_copy(
                h_hbm.at[didx.at[k]],
                hd_blk.at[pl.ds(k * CHUNK, CHUNK)], sem))
        for cp in cps:
            cp.wait()
        pltpu.sync_copy(hs_blk, hs_out.at[pl.ds(ebase, GEDGES)])
        pltpu.sync_copy(hd_blk, hd_out.at[pl.ds(ebase, GEDGES)])
        return carry

    lax.fori_loop(0, NGRP_G, body, 0)


@functools.cache
def _gather_edges_kernel():
    return pl.kernel(
        _gather_body,
        out_type=[jax.ShapeDtypeStruct((EP, NODE_EMB), jnp.float32),
                  jax.ShapeDtypeStruct((EP, NODE_EMB), jnp.float32)],
        scratch_types=[
            pltpu.VMEM((GG, CHUNK), jnp.int32),
            pltpu.VMEM((GG, CHUNK), jnp.int32),
            pltpu.VMEM((GEDGES, NODE_EMB), jnp.float32),
            pltpu.VMEM((GEDGES, NODE_EMB), jnp.float32),
            pltpu.SemaphoreType.DMA,
        ],
        mesh=_sc_mesh(),
        compiler_params=_SC_PARAMS,
    )


def _scatter_body(msg_hbm, dst2_hbm, h_hbm, out_hbm,
                  idx_v, buf_v, stage_v, acc, sem):
    c = lax.axis_index("c")
    s = lax.axis_index("s")
    r0 = s * ROWS_PT

    for p in range(2):
        col = c * HALF + p * QCOL

        pltpu.sync_copy(h_hbm.at[pl.ds(r0, ROWS_PT), pl.ds(col, QCOL)],
                        stage_v)
        pltpu.sync_copy(stage_v, acc.at[pl.ds(r0, ROWS_PT)])
        plsc.subcore_barrier()

        def body(g, carry):
            cbase = s * (EPT // CHUNK) + g * GS
            ebase = s * EPT + g * SEDGES
            pltpu.sync_copy(dst2_hbm.at[pl.ds(cbase, GS)], idx_v)
            pltpu.sync_copy(
                msg_hbm.at[pl.ds(ebase, SEDGES), pl.ds(col, QCOL)], buf_v)
            cps = []
            for k in range(GS):
                cps.append(pltpu.async_copy(
                    buf_v.at[pl.ds(k * CHUNK, CHUNK)],
                    acc.at[idx_v.at[k]], sem, add=True))
            for cp in cps:
                cp.wait()
            return carry

        lax.fori_loop(0, NGRP_S, body, 0)
        plsc.subcore_barrier()

        pltpu.sync_copy(acc.at[pl.ds(r0, ROWS_PT)], stage_v)
        pltpu.sync_copy(stage_v,
                        out_hbm.at[pl.ds(r0, ROWS_PT), pl.ds(col, QCOL)])


@functools.cache
def _segment_add_kernel():
    return pl.kernel(
        _scatter_body,
        out_type=jax.ShapeDtypeStruct((NP, NODE_EMB), jnp.float32),
        scratch_types=[
            pltpu.VMEM((GS, CHUNK), jnp.int32),
            pltpu.VMEM((SEDGES, QCOL), jnp.float32),
            pltpu.VMEM((ROWS_PT, QCOL), jnp.float32),
            pltpu.VMEM_SHARED((NP, QCOL), jnp.float32),
            pltpu.SemaphoreType.DMA,
        ],
        mesh=_sc_mesh(),
        compiler_params=_SC_PARAMS,
    )


def _cgather_body(table_hbm, cidx_hbm, out_hbm, table_v, cidx_v, out_v):
    c = lax.axis_index("c")
    s = lax.axis_index("s")
    wid = s * NC + c
    pltpu.sync_copy(table_hbm, table_v)
    pltpu.sync_copy(cidx_hbm.at[pl.ds(wid * CPW, CPW)], cidx_v)
    for k in range(CSTEPS):
        iv = cidx_v[pl.ds(k * 16, 16)]
        out_v[pl.ds(k * 16, 16)] = plsc.load_gather(table_v, [iv])
    pltpu.sync_copy(out_v, out_hbm.at[pl.ds(wid * CPW, CPW)])


@functools.cache
def _cgather_kernel(dtype):
    return pl.kernel(
        _cgather_body,
        out_type=jax.ShapeDtypeStruct((CPAD,), dtype),
        scratch_types=[
            pltpu.VMEM((NP,), dtype),
            pltpu.VMEM((CPW,), jnp.int32),
            pltpu.VMEM((CPW,), dtype),
        ],
        mesh=_sc_mesh(),
        compiler_params=_SC_PARAMS,
    )


def _embed_node_body(x_ref, w_ref, b_ref, o_ref):
    x = x_ref[...]
    w = w_ref[...]
    o_ref[...] = (x[:, 0:1] * w[0:1, :] + x[:, 1:2] * w[1:2, :] + b_ref[...])


def _embed_edge_body(a_ref, w_ref, b_ref, o_ref):
    o_ref[...] = (jnp.dot(a_ref[...], w_ref[...],
                          preferred_element_type=jnp.float32) + b_ref[...])


NBLK = 12512
EBLK = 6400
EVB = E // EBLK - 1


def _embed_nodes(x, w, b):
    return pl.pallas_call(
        _embed_node_body,
        grid=(NP // NBLK,),
        in_specs=[pl.BlockSpec((NBLK, IN_DIM), lambda i: (i, 0)),
                  pl.BlockSpec((IN_DIM, NODE_EMB), lambda i: (0, 0)),
                  pl.BlockSpec((1, NODE_EMB), lambda i: (0, 0))],
        out_specs=pl.BlockSpec((NBLK, NODE_EMB), lambda i: (i, 0)),
        out_shape=jax.ShapeDtypeStruct((NP, NODE_EMB), jnp.float32),
    )(x, w, b)


def _embed_edges(a, w, b):
    return pl.pallas_call(
        _embed_edge_body,
        grid=(EP // EBLK,),
        in_specs=[pl.BlockSpec((EBLK, EDGE_DIM),
                               lambda i: (jnp.minimum(i, EVB), 0)),
                  pl.BlockSpec((EDGE_DIM, EDGE_EMB), lambda i: (0, 0)),
                  pl.BlockSpec((1, EDGE_EMB), lambda i: (0, 0))],
        out_specs=pl.BlockSpec((EBLK, EDGE_EMB), lambda i: (i, 0)),
        out_shape=jax.ShapeDtypeStruct((EP, EDGE_EMB), jnp.float32),
    )(a, w, b)


def _edge_mlp_body(hs_ref, hd_ref, e_ref, w_ref, b_ref, msg_ref, eout_ref):
    e = e_ref[...]
    feat = jnp.concatenate([hs_ref[...], hd_ref[...], e], axis=-1)
    r = jnp.dot(feat, w_ref[...], preferred_element_type=jnp.float32)
    r = jnp.maximum(r + b_ref[...], 0.0)
    msg_ref[...] = r[:, :NODE_EMB]
    eout_ref[...] = e + r[:, NODE_EMB:]


def _edge_mlp(hs, hd, e, w, b):
    return pl.pallas_call(
        _edge_mlp_body,
        grid=(EP // EBLK,),
        in_specs=[pl.BlockSpec((EBLK, NODE_EMB), lambda i: (i, 0)),
                  pl.BlockSpec((EBLK, NODE_EMB), lambda i: (i, 0)),
                  pl.BlockSpec((EBLK, EDGE_EMB), lambda i: (i, 0)),
                  pl.BlockSpec((CAT_DIM, OUT_DIM), lambda i: (0, 0)),
                  pl.BlockSpec((1, OUT_DIM), lambda i: (0, 0))],
        out_specs=[pl.BlockSpec((EBLK, NODE_EMB), lambda i: (i, 0)),
                   pl.BlockSpec((EBLK, EDGE_EMB), lambda i: (i, 0))],
        out_shape=[jax.ShapeDtypeStruct((EP, NODE_EMB), jnp.float32),
                   jax.ShapeDtypeStruct((EP, EDGE_EMB), jnp.float32)],
    )(hs, hd, e, w, b)


def _score_body(h_ref, w_ref, b_ref, o_ref):
    o_ref[...] = (jnp.sum(h_ref[...] * w_ref[...], axis=1, keepdims=True)
                  + b_ref[...])


def _node_scores(h, w_row, b):
    return pl.pallas_call(
        _score_body,
        grid=(NP // NBLK,),
        in_specs=[pl.BlockSpec((NBLK, NODE_EMB), lambda i: (i, 0)),
                  pl.BlockSpec((1, NODE_EMB), lambda i: (0, 0)),
                  pl.BlockSpec((1, 1), lambda i: (0, 0))],
        out_specs=pl.BlockSpec((NBLK, 1), lambda i: (i, 0)),
        out_shape=jax.ShapeDtypeStruct((NP, 1), jnp.float32),
    )(h, w_row, b)


def _softmax_body(cl_ref, seg_ref, out_ref):
    cl = cl_ref[...]
    seg = seg_ref[...]
    gi = lax.broadcasted_iota(jnp.int32, (NUM_GRAPHS, CPAD), 0)
    ci = lax.broadcasted_iota(jnp.int32, (NUM_GRAPHS, CPAD), 1)
    eq = (seg[None, :] == gi) & (ci < NUM_CAND)
    neg = jnp.full((), -jnp.inf, jnp.float32)
    m = jnp.max(jnp.where(eq, cl[None, :], neg), axis=1)
    m = jnp.where(jnp.isfinite(m), m, 0.0)
    msel = jnp.sum(jnp.where(eq, m[:, None], 0.0), axis=0)
    shifted = cl - msel
    se = jnp.sum(jnp.where(eq, jnp.exp(shifted)[None, :], 0.0), axis=1)
    lsel = jnp.sum(jnp.where(eq, jnp.log(se)[:, None], 0.0), axis=0)
    out_ref[...] = shifted - lsel


def _group_log_softmax(cl, seg):
    return pl.pallas_call(
        _softmax_body,
        out_shape=jax.ShapeDtypeStruct((CPAD,), jnp.float32),
    )(cl, seg)


def kernel(x, edge_attr, edge_index, candidate_idxs, batch,
           Wn_in, bn_in, We_in, be_in, msg_W, msg_b, edge_W, edge_b,
           Wout, bout):
    pad_e = EP - E
    src2 = jnp.concatenate(
        [edge_index[0], jnp.zeros((pad_e,), jnp.int32)]
    ).reshape(EP // CHUNK, CHUNK)
    dst2 = jnp.concatenate(
        [edge_index[1], N + (jnp.arange(pad_e, dtype=jnp.int32) % (NP - N))]
    ).reshape(EP // CHUNK, CHUNK)
    xp = jnp.concatenate([x, jnp.zeros((NP - N, IN_DIM), x.dtype)])
    batchp = jnp.concatenate([batch, jnp.zeros((NP - N,), jnp.int32)])
    w_all = jnp.concatenate([msg_W, edge_W], axis=-1)
    b_all = jnp.concatenate([msg_b, edge_b], axis=-1)

    h = _embed_nodes(xp, Wn_in, bn_in.reshape(1, NODE_EMB))
    e = _embed_edges(edge_attr, We_in, be_in.reshape(1, EDGE_EMB))

    def layer(carry, wb):
        h, e = carry
        w, b = wb
        hs, hd = _gather_edges_kernel()(h, src2, dst2)
        msg, e2 = _edge_mlp(hs, hd, e, w, b)
        h2 = _segment_add_kernel()(msg, dst2, h)
        return (h2, e2), None

    (h, e), _ = lax.scan(
        layer, (h, e), (w_all, b_all.reshape(LAYERS, 1, OUT_DIM)))

    score = _node_scores(h, Wout.reshape(1, NODE_EMB),
                         bout.reshape(1, 1)).reshape(NP)
    cpad = jnp.concatenate(
        [candidate_idxs, jnp.zeros((CPAD - NUM_CAND,), jnp.int32)])
    cl = _cgather_kernel(jnp.float32)(score, cpad)
    seg = _cgather_kernel(jnp.int32)(batchp, cpad)
    out = _group_log_softmax(cl, seg)
    return out[:NUM_CAND]


# speedup vs baseline: 1.1225x; 1.1225x over previous
import functools

import jax
import jax.numpy as jnp
from jax import lax
from jax.experimental import pallas as pl
from jax.experimental.pallas import tpu as pltpu
from jax.experimental.pallas import tpu_sc as plsc

N = 100000
E = 1600000
IN_DIM = 2
EDGE_DIM = 16
NODE_EMB = 32
EDGE_EMB = 8
LAYERS = 4
NUM_CAND = 10000
NUM_GRAPHS = 64
CAT_DIM = 2 * NODE_EMB + EDGE_EMB
OUT_DIM = NODE_EMB + EDGE_EMB

NC = 2
NS = 16
NW = NC * NS

CHUNK = 80
EPW = E // NW
GG = 5
GEDGES = GG * CHUNK
NGRP_G = EPW // GEDGES
EPT = E // NS
GS = 10
SEDGES = GS * CHUNK
NGRP_S = EPT // SEDGES
ROWS_PT = N // NS
HALF = NODE_EMB // NC
QCOL = HALF // 2

CPAD = 10240
CPW = CPAD // NW
CSTEPS = CPW // 16


def _sc_mesh():
    return plsc.VectorSubcoreMesh(core_axis_name="c", subcore_axis_name="s")


_SC_PARAMS = pltpu.CompilerParams(use_tc_tiling_on_sc=False,
                                  needs_layout_passes=False)


def _gather_body(h_hbm, src2_hbm, dst2_hbm, hs_out, hd_out,
                 sidx, didx, hs_blk, hd_blk, sem):
    c = lax.axis_index("c")
    s = lax.axis_index("s")
    wid = s * NC + c

    def body(g, carry):
        cbase = wid * (EPW // CHUNK) + g * GG
        ebase = wid * EPW + g * GEDGES
        pltpu.sync_copy(src2_hbm.at[pl.ds(cbase, GG)], sidx)
        pltpu.sync_copy(dst2_hbm.at[pl.ds(cbase, GG)], didx)
        cps = []
        for k in range(GG):
            cps.append(pltpu.async_copy(
                h_hbm.at[sidx.at[k]],
                hs_blk.at[pl.ds(k * CHUNK, CHUNK)], sem))
            cps.append(pltpu.async_copy(
                h_hbm.at[didx.at[k]],
                hd_blk.at[pl.ds(k * CHUNK, CHUNK)], sem))
        for cp in cps:
            cp.wait()
        pltpu.sync_copy(hs_blk, hs_out.at[pl.ds(ebase, GEDGES)])
        pltpu.sync_copy(hd_blk, hd_out.at[pl.ds(ebase, GEDGES)])
        return carry

    lax.fori_loop(0, NGRP_G, body, 0)


@functools.cache
def _gather_edges_kernel():
    return pl.kernel(
        _gather_body,
        out_type=[jax.ShapeDtypeStruct((E, NODE_EMB), jnp.float32),
                  jax.ShapeDtypeStruct((E, NODE_EMB), jnp.float32)],
        scratch_types=[
            pltpu.VMEM((GG, CHUNK), jnp.int32),
            pltpu.VMEM((GG, CHUNK), jnp.int32),
            pltpu.VMEM((GEDGES, NODE_EMB), jnp.float32),
            pltpu.VMEM((GEDGES, NODE_EMB), jnp.float32),
            pltpu.SemaphoreType.DMA,
        ],
        mesh=_sc_mesh(),
        compiler_params=_SC_PARAMS,
    )


def _scatter_body(msg_hbm, dst2_hbm, h_hbm, out_hbm,
                  idx_v, buf_v, stage_v, acc, sem):
    c = lax.axis_index("c")
    s = lax.axis_index("s")
    r0 = s * ROWS_PT

    for p in range(2):
        col = c * HALF + p * QCOL

        pltpu.sync_copy(h_hbm.at[pl.ds(r0, ROWS_PT), pl.ds(col, QCOL)],
                        stage_v)
        pltpu.sync_copy(stage_v, acc.at[pl.ds(r0, ROWS_PT)])
        plsc.subcore_barrier()

        def body(g, carry):
            cbase = s * (EPT // CHUNK) + g * GS
            ebase = s * EPT + g * SEDGES
            pltpu.sync_copy(dst2_hbm.at[pl.ds(cbase, GS)], idx_v)
            pltpu.sync_copy(
                msg_hbm.at[pl.ds(ebase, SEDGES), pl.ds(col, QCOL)], buf_v)
            cps = []
            for k in range(GS):
                cps.append(pltpu.async_copy(
                    buf_v.at[pl.ds(k * CHUNK, CHUNK)],
                    acc.at[idx_v.at[k]], sem, add=True))
            for cp in cps:
                cp.wait()
            return carry

        lax.fori_loop(0, NGRP_S, body, 0)
        plsc.subcore_barrier()

        pltpu.sync_copy(acc.at[pl.ds(r0, ROWS_PT)], stage_v)
        pltpu.sync_copy(stage_v,
                        out_hbm.at[pl.ds(r0, ROWS_PT), pl.ds(col, QCOL)])


@functools.cache
def _segment_add_kernel():
    return pl.kernel(
        _scatter_body,
        out_type=jax.ShapeDtypeStruct((N, NODE_EMB), jnp.float32),
        scratch_types=[
            pltpu.VMEM((GS, CHUNK), jnp.int32),
            pltpu.VMEM((SEDGES, QCOL), jnp.float32),
            pltpu.VMEM((ROWS_PT, QCOL), jnp.float32),
            pltpu.VMEM_SHARED((N, QCOL), jnp.float32),
            pltpu.SemaphoreType.DMA,
        ],
        mesh=_sc_mesh(),
        compiler_params=_SC_PARAMS,
    )


def _cgather_body(table_hbm, cidx_hbm, out_hbm, table_v, cidx_v, out_v):
    c = lax.axis_index("c")
    s = lax.axis_index("s")
    wid = s * NC + c
    pltpu.sync_copy(table_hbm, table_v)
    pltpu.sync_copy(cidx_hbm.at[pl.ds(wid * CPW, CPW)], cidx_v)
    for k in range(CSTEPS):
        iv = cidx_v[pl.ds(k * 16, 16)]
        out_v[pl.ds(k * 16, 16)] = plsc.load_gather(table_v, [iv])
    pltpu.sync_copy(out_v, out_hbm.at[pl.ds(wid * CPW, CPW)])


@functools.cache
def _cgather_kernel(dtype):
    return pl.kernel(
        _cgather_body,
        out_type=jax.ShapeDtypeStruct((CPAD,), dtype),
        scratch_types=[
            pltpu.VMEM((N,), dtype),
            pltpu.VMEM((CPW,), jnp.int32),
            pltpu.VMEM((CPW,), dtype),
        ],
        mesh=_sc_mesh(),
        compiler_params=_SC_PARAMS,
    )


def _embed_node_body(x_ref, w_ref, b_ref, o_ref):
    x = x_ref[...]
    w = w_ref[...]
    o_ref[...] = (x[:, 0:1] * w[0:1, :] + x[:, 1:2] * w[1:2, :] + b_ref[...])


def _embed_edge_body(a_ref, w_ref, b_ref, o_ref):
    o_ref[...] = (jnp.dot(a_ref[...], w_ref[...],
                          preferred_element_type=jnp.float32) + b_ref[...])


NBLK = 10000
EBLK = 8000


def _embed_nodes(x, w, b):
    return pl.pallas_call(
        _embed_node_body,
        grid=(N // NBLK,),
        in_specs=[pl.BlockSpec((NBLK, IN_DIM), lambda i: (i, 0)),
                  pl.BlockSpec((IN_DIM, NODE_EMB), lambda i: (0, 0)),
                  pl.BlockSpec((1, NODE_EMB), lambda i: (0, 0))],
        out_specs=pl.BlockSpec((NBLK, NODE_EMB), lambda i: (i, 0)),
        out_shape=jax.ShapeDtypeStruct((N, NODE_EMB), jnp.float32),
    )(x, w, b)


def _embed_edges(a, w, b):
    return pl.pallas_call(
        _embed_edge_body,
        grid=(E // EBLK,),
        in_specs=[pl.BlockSpec((EBLK, EDGE_DIM), lambda i: (i, 0)),
                  pl.BlockSpec((EDGE_DIM, EDGE_EMB), lambda i: (0, 0)),
                  pl.BlockSpec((1, EDGE_EMB), lambda i: (0, 0))],
        out_specs=pl.BlockSpec((EBLK, EDGE_EMB), lambda i: (i, 0)),
        out_shape=jax.ShapeDtypeStruct((E, EDGE_EMB), jnp.float32),
    )(a, w, b)


def _edge_mlp_body(hs_ref, hd_ref, e_ref, w_ref, b_ref, msg_ref, eout_ref):
    e = e_ref[...]
    feat = jnp.concatenate([hs_ref[...], hd_ref[...], e], axis=-1)
    r = jnp.dot(feat, w_ref[...], preferred_element_type=jnp.float32)
    r = jnp.maximum(r + b_ref[...], 0.0)
    msg_ref[...] = r[:, :NODE_EMB]
    eout_ref[...] = e + r[:, NODE_EMB:]


def _edge_mlp(hs, hd, e, w, b):
    return pl.pallas_call(
        _edge_mlp_body,
        grid=(E // EBLK,),
        in_specs=[pl.BlockSpec((EBLK, NODE_EMB), lambda i: (i, 0)),
                  pl.BlockSpec((EBLK, NODE_EMB), lambda i: (i, 0)),
                  pl.BlockSpec((EBLK, EDGE_EMB), lambda i: (i, 0)),
                  pl.BlockSpec((CAT_DIM, OUT_DIM), lambda i: (0, 0)),
                  pl.BlockSpec((1, OUT_DIM), lambda i: (0, 0))],
        out_specs=[pl.BlockSpec((EBLK, NODE_EMB), lambda i: (i, 0)),
                   pl.BlockSpec((EBLK, EDGE_EMB), lambda i: (i, 0))],
        out_shape=[jax.ShapeDtypeStruct((E, NODE_EMB), jnp.float32),
                   jax.ShapeDtypeStruct((E, EDGE_EMB), jnp.float32)],
    )(hs, hd, e, w, b)


def _score_body(h_ref, w_ref, b_ref, o_ref):
    o_ref[...] = (jnp.sum(h_ref[...] * w_ref[...], axis=1, keepdims=True)
                  + b_ref[...])


def _node_scores(h, w_row, b):
    return pl.pallas_call(
        _score_body,
        grid=(N // NBLK,),
        in_specs=[pl.BlockSpec((NBLK, NODE_EMB), lambda i: (i, 0)),
                  pl.BlockSpec((1, NODE_EMB), lambda i: (0, 0)),
                  pl.BlockSpec((1, 1), lambda i: (0, 0))],
        out_specs=pl.BlockSpec((NBLK, 1), lambda i: (i, 0)),
        out_shape=jax.ShapeDtypeStruct((N, 1), jnp.float32),
    )(h, w_row, b)


def _softmax_body(cl_ref, seg_ref, out_ref):
    cl = cl_ref[...]
    seg = seg_ref[...]
    gi = lax.broadcasted_iota(jnp.int32, (NUM_GRAPHS, CPAD), 0)
    ci = lax.broadcasted_iota(jnp.int32, (NUM_GRAPHS, CPAD), 1)
    eq = (seg[None, :] == gi) & (ci < NUM_CAND)
    neg = jnp.full((), -jnp.inf, jnp.float32)
    m = jnp.max(jnp.where(eq, cl[None, :], neg), axis=1)
    m = jnp.where(jnp.isfinite(m), m, 0.0)
    msel = jnp.sum(jnp.where(eq, m[:, None], 0.0), axis=0)
    shifted = cl - msel
    se = jnp.sum(jnp.where(eq, jnp.exp(shifted)[None, :], 0.0), axis=1)
    lsel = jnp.sum(jnp.where(eq, jnp.log(se)[:, None], 0.0), axis=0)
    out_ref[...] = shifted - lsel


def _group_log_softmax(cl, seg):
    return pl.pallas_call(
        _softmax_body,
        out_shape=jax.ShapeDtypeStruct((CPAD,), jnp.float32),
    )(cl, seg)


def kernel(x, edge_attr, edge_index, candidate_idxs, batch,
           Wn_in, bn_in, We_in, be_in, msg_W, msg_b, edge_W, edge_b,
           Wout, bout):
    src2 = edge_index[0].reshape(E // CHUNK, CHUNK)
    dst2 = edge_index[1].reshape(E // CHUNK, CHUNK)
    w_all = jnp.concatenate([msg_W, edge_W], axis=-1)
    b_all = jnp.concatenate([msg_b, edge_b], axis=-1)

    h = _embed_nodes(x, Wn_in, bn_in.reshape(1, NODE_EMB))
    e = _embed_edges(edge_attr, We_in, be_in.reshape(1, EDGE_EMB))

    def layer(carry, wb):
        h, e = carry
        w, b = wb
        hs, hd = _gather_edges_kernel()(h, src2, dst2)
        msg, e2 = _edge_mlp(hs, hd, e, w, b)
        h2 = _segment_add_kernel()(msg, dst2, h)
        return (h2, e2), None

    (h, e), _ = lax.scan(
        layer, (h, e), (w_all, b_all.reshape(LAYERS, 1, OUT_DIM)))

    score = _node_scores(h, Wout.reshape(1, NODE_EMB),
                         bout.reshape(1, 1)).reshape(N)
    cpad = jnp.concatenate(
        [candidate_idxs, jnp.zeros((CPAD - NUM_CAND,), jnp.int32)])
    cl = _cgather_kernel(jnp.float32)(score, cpad)
    seg = _cgather_kernel(jnp.int32)(batch, cpad)
    out = _group_log_softmax(cl, seg)
    return out[:NUM_CAND]


# speedup vs baseline: 1.1703x; 1.0425x over previous
import functools

import jax
import jax.numpy as jnp
from jax import lax
from jax.experimental import pallas as pl
from jax.experimental.pallas import tpu as pltpu
from jax.experimental.pallas import tpu_sc as plsc

N = 100000
E = 1600000
IN_DIM = 2
EDGE_DIM = 16
NODE_EMB = 32
EDGE_EMB = 8
LAYERS = 4
NUM_CAND = 10000
NUM_GRAPHS = 64
CAT_DIM = 2 * NODE_EMB + EDGE_EMB
OUT_DIM = NODE_EMB + EDGE_EMB

NC = 2
NS = 16
NW = NC * NS

CHUNK = 80
EPW = E // NW
GG = 5
GEDGES = GG * CHUNK
NGRP_G = EPW // GEDGES
EPT = E // NS
GS = 10
SEDGES = GS * CHUNK
NGRP_S = EPT // SEDGES
ROWS_PT = N // NS
HALF = NODE_EMB // NC
QCOL = HALF // 2

CPAD = 10240
CPW = CPAD // NW
CSTEPS = CPW // 16


def _sc_mesh():
    return plsc.VectorSubcoreMesh(core_axis_name="c", subcore_axis_name="s")


_SC_PARAMS = pltpu.CompilerParams(use_tc_tiling_on_sc=False,
                                  needs_layout_passes=False)


def _gather_body(h_hbm, src2_hbm, dst2_hbm, hs_out, hd_out,
                 sidx, didx, hs_blk, hd_blk, sem):
    c = lax.axis_index("c")
    s = lax.axis_index("s")
    wid = s * NC + c

    def body(g, carry):
        cbase = wid * (EPW // CHUNK) + g * GG
        ebase = wid * EPW + g * GEDGES
        ia = pltpu.async_copy(src2_hbm.at[pl.ds(cbase, GG)], sidx, sem)
        ib = pltpu.async_copy(dst2_hbm.at[pl.ds(cbase, GG)], didx, sem)
        ia.wait()
        ib.wait()
        cps = []
        for k in range(GG):
            cps.append(pltpu.async_copy(
                h_hbm.at[sidx.at[k]],
                hs_blk.at[pl.ds(k * CHUNK, CHUNK)], sem))
            cps.append(pltpu.async_copy(
                h_hbm.at[didx.at[k]],
                hd_blk.at[pl.ds(k * CHUNK, CHUNK)], sem))
        for cp in cps:
            cp.wait()
        wa = pltpu.async_copy(hs_blk, hs_out.at[pl.ds(ebase, GEDGES)], sem)
        wb = pltpu.async_copy(hd_blk, hd_out.at[pl.ds(ebase, GEDGES)], sem)
        wa.wait()
        wb.wait()
        return carry

    lax.fori_loop(0, NGRP_G, body, 0)


@functools.cache
def _gather_edges_kernel():
    return pl.kernel(
        _gather_body,
        out_type=[jax.ShapeDtypeStruct((E, NODE_EMB), jnp.float32),
                  jax.ShapeDtypeStruct((E, NODE_EMB), jnp.float32)],
        scratch_types=[
            pltpu.VMEM((GG, CHUNK), jnp.int32),
            pltpu.VMEM((GG, CHUNK), jnp.int32),
            pltpu.VMEM((GEDGES, NODE_EMB), jnp.float32),
            pltpu.VMEM((GEDGES, NODE_EMB), jnp.float32),
            pltpu.SemaphoreType.DMA,
        ],
        mesh=_sc_mesh(),
        compiler_params=_SC_PARAMS,
    )


def _scatter_body(msg_hbm, dst2_hbm, h_hbm, out_hbm,
                  idx_v, buf_v, stage_v, acc, sem):
    c = lax.axis_index("c")
    s = lax.axis_index("s")
    r0 = s * ROWS_PT

    for p in range(2):
        col = c * HALF + p * QCOL

        pltpu.sync_copy(h_hbm.at[pl.ds(r0, ROWS_PT), pl.ds(col, QCOL)],
                        stage_v)
        pltpu.sync_copy(stage_v, acc.at[pl.ds(r0, ROWS_PT)])
        plsc.subcore_barrier()

        def body(g, carry):
            cbase = s * (EPT // CHUNK) + g * GS
            ebase = s * EPT + g * SEDGES
            ia = pltpu.async_copy(dst2_hbm.at[pl.ds(cbase, GS)], idx_v,
                                  sem)
            ib = pltpu.async_copy(
                msg_hbm.at[pl.ds(ebase, SEDGES), pl.ds(col, QCOL)], buf_v,
                sem)
            ia.wait()
            ib.wait()
            cps = []
            for k in range(GS):
                cps.append(pltpu.async_copy(
                    buf_v.at[pl.ds(k * CHUNK, CHUNK)],
                    acc.at[idx_v.at[k]], sem, add=True))
            for cp in cps:
                cp.wait()
            return carry

        lax.fori_loop(0, NGRP_S, body, 0)
        plsc.subcore_barrier()

        pltpu.sync_copy(acc.at[pl.ds(r0, ROWS_PT)], stage_v)
        pltpu.sync_copy(stage_v,
                        out_hbm.at[pl.ds(r0, ROWS_PT), pl.ds(col, QCOL)])


@functools.cache
def _segment_add_kernel():
    return pl.kernel(
        _scatter_body,
        out_type=jax.ShapeDtypeStruct((N, NODE_EMB), jnp.float32),
        scratch_types=[
            pltpu.VMEM((GS, CHUNK), jnp.int32),
            pltpu.VMEM((SEDGES, QCOL), jnp.float32),
            pltpu.VMEM((ROWS_PT, QCOL), jnp.float32),
            pltpu.VMEM_SHARED((N, QCOL), jnp.float32),
            pltpu.SemaphoreType.DMA,
        ],
        mesh=_sc_mesh(),
        compiler_params=_SC_PARAMS,
    )


def _cgather_body(table_hbm, cidx_hbm, out_hbm, table_v, cidx_v, out_v):
    c = lax.axis_index("c")
    s = lax.axis_index("s")
    wid = s * NC + c
    pltpu.sync_copy(table_hbm, table_v)
    pltpu.sync_copy(cidx_hbm.at[pl.ds(wid * CPW, CPW)], cidx_v)
    for k in range(CSTEPS):
        iv = cidx_v[pl.ds(k * 16, 16)]
        out_v[pl.ds(k * 16, 16)] = plsc.load_gather(table_v, [iv])
    pltpu.sync_copy(out_v, out_hbm.at[pl.ds(wid * CPW, CPW)])


@functools.cache
def _cgather_kernel(dtype):
    return pl.kernel(
        _cgather_body,
        out_type=jax.ShapeDtypeStruct((CPAD,), dtype),
        scratch_types=[
            pltpu.VMEM((N,), dtype),
            pltpu.VMEM((CPW,), jnp.int32),
            pltpu.VMEM((CPW,), dtype),
        ],
        mesh=_sc_mesh(),
        compiler_params=_SC_PARAMS,
    )


def _embed_node_body(x_ref, w_ref, b_ref, o_ref):
    x = x_ref[...]
    w = w_ref[...]
    o_ref[...] = (x[:, 0:1] * w[0:1, :] + x[:, 1:2] * w[1:2, :] + b_ref[...])


def _embed_edge_body(a_ref, w_ref, b_ref, o_ref):
    o_ref[...] = (jnp.dot(a_ref[...], w_ref[...],
                          preferred_element_type=jnp.float32) + b_ref[...])


NBLK = 10000
EBLK = 8000


def _embed_nodes(x, w, b):
    return pl.pallas_call(
        _embed_node_body,
        grid=(N // NBLK,),
        in_specs=[pl.BlockSpec((NBLK, IN_DIM), lambda i: (i, 0)),
                  pl.BlockSpec((IN_DIM, NODE_EMB), lambda i: (0, 0)),
                  pl.BlockSpec((1, NODE_EMB), lambda i: (0, 0))],
        out_specs=pl.BlockSpec((NBLK, NODE_EMB), lambda i: (i, 0)),
        out_shape=jax.ShapeDtypeStruct((N, NODE_EMB), jnp.float32),
    )(x, w, b)


def _embed_edges(a, w, b):
    return pl.pallas_call(
        _embed_edge_body,
        grid=(E // EBLK,),
        in_specs=[pl.BlockSpec((EBLK, EDGE_DIM), lambda i: (i, 0)),
                  pl.BlockSpec((EDGE_DIM, EDGE_EMB), lambda i: (0, 0)),
                  pl.BlockSpec((1, EDGE_EMB), lambda i: (0, 0))],
        out_specs=pl.BlockSpec((EBLK, EDGE_EMB), lambda i: (i, 0)),
        out_shape=jax.ShapeDtypeStruct((E, EDGE_EMB), jnp.float32),
    )(a, w, b)


def _edge_mlp_body(hs_ref, hd_ref, e_ref, w_ref, b_ref, msg_ref, eout_ref):
    e = e_ref[...]
    feat = jnp.concatenate([hs_ref[...], hd_ref[...], e], axis=-1)
    r = jnp.dot(feat, w_ref[...], preferred_element_type=jnp.float32)
    r = jnp.maximum(r + b_ref[...], 0.0)
    msg_ref[...] = r[:, :NODE_EMB]
    eout_ref[...] = e + r[:, NODE_EMB:]


def _edge_mlp(hs, hd, e, w, b):
    return pl.pallas_call(
        _edge_mlp_body,
        grid=(E // EBLK,),
        in_specs=[pl.BlockSpec((EBLK, NODE_EMB), lambda i: (i, 0)),
                  pl.BlockSpec((EBLK, NODE_EMB), lambda i: (i, 0)),
                  pl.BlockSpec((EBLK, EDGE_EMB), lambda i: (i, 0)),
                  pl.BlockSpec((CAT_DIM, OUT_DIM), lambda i: (0, 0)),
                  pl.BlockSpec((1, OUT_DIM), lambda i: (0, 0))],
        out_specs=[pl.BlockSpec((EBLK, NODE_EMB), lambda i: (i, 0)),
                   pl.BlockSpec((EBLK, EDGE_EMB), lambda i: (i, 0))],
        out_shape=[jax.ShapeDtypeStruct((E, NODE_EMB), jnp.float32),
                   jax.ShapeDtypeStruct((E, EDGE_EMB), jnp.float32)],
    )(hs, hd, e, w, b)


def _score_body(h_ref, w_ref, b_ref, o_ref):
    o_ref[...] = (jnp.sum(h_ref[...] * w_ref[...], axis=1, keepdims=True)
                  + b_ref[...])


def _node_scores(h, w_row, b):
    return pl.pallas_call(
        _score_body,
        grid=(N // NBLK,),
        in_specs=[pl.BlockSpec((NBLK, NODE_EMB), lambda i: (i, 0)),
                  pl.BlockSpec((1, NODE_EMB), lambda i: (0, 0)),
                  pl.BlockSpec((1, 1), lambda i: (0, 0))],
        out_specs=pl.BlockSpec((NBLK, 1), lambda i: (i, 0)),
        out_shape=jax.ShapeDtypeStruct((N, 1), jnp.float32),
    )(h, w_row, b)


def _softmax_body(cl_ref, seg_ref, out_ref):
    cl = cl_ref[...]
    seg = seg_ref[...]
    gi = lax.broadcasted_iota(jnp.int32, (NUM_GRAPHS, CPAD), 0)
    ci = lax.broadcasted_iota(jnp.int32, (NUM_GRAPHS, CPAD), 1)
    eq = (seg[None, :] == gi) & (ci < NUM_CAND)
    neg = jnp.full((), -jnp.inf, jnp.float32)
    m = jnp.max(jnp.where(eq, cl[None, :], neg), axis=1)
    m = jnp.where(jnp.isfinite(m), m, 0.0)
    msel = jnp.sum(jnp.where(eq, m[:, None], 0.0), axis=0)
    shifted = cl - msel
    se = jnp.sum(jnp.where(eq, jnp.exp(shifted)[None, :], 0.0), axis=1)
    lsel = jnp.sum(jnp.where(eq, jnp.log(se)[:, None], 0.0), axis=0)
    out_ref[...] = shifted - lsel


def _group_log_softmax(cl, seg):
    return pl.pallas_call(
        _softmax_body,
        out_shape=jax.ShapeDtypeStruct((CPAD,), jnp.float32),
    )(cl, seg)


def kernel(x, edge_attr, edge_index, candidate_idxs, batch,
           Wn_in, bn_in, We_in, be_in, msg_W, msg_b, edge_W, edge_b,
           Wout, bout):
    src2 = edge_index[0].reshape(E // CHUNK, CHUNK)
    dst2 = edge_index[1].reshape(E // CHUNK, CHUNK)
    w_all = jnp.concatenate([msg_W, edge_W], axis=-1)
    b_all = jnp.concatenate([msg_b, edge_b], axis=-1)

    h = _embed_nodes(x, Wn_in, bn_in.reshape(1, NODE_EMB))
    e = _embed_edges(edge_attr, We_in, be_in.reshape(1, EDGE_EMB))

    def layer(carry, wb):
        h, e = carry
        w, b = wb
        hs, hd = _gather_edges_kernel()(h, src2, dst2)
        msg, e2 = _edge_mlp(hs, hd, e, w, b)
        h2 = _segment_add_kernel()(msg, dst2, h)
        return (h2, e2), None

    (h, e), _ = lax.scan(
        layer, (h, e), (w_all, b_all.reshape(LAYERS, 1, OUT_DIM)))

    score = _node_scores(h, Wout.reshape(1, NODE_EMB),
                         bout.reshape(1, 1)).reshape(N)
    cpad = jnp.concatenate(
        [candidate_idxs, jnp.zeros((CPAD - NUM_CAND,), jnp.int32)])
    cl = _cgather_kernel(jnp.float32)(score, cpad)
    seg = _cgather_kernel(jnp.int32)(batch, cpad)
    out = _group_log_softmax(cl, seg)
    return out[:NUM_CAND]
